# baseline (speedup 1.0000x reference)
import functools

import jax
import jax.numpy as jnp
from jax import lax
from jax.experimental import pallas as pl
from jax.experimental.pallas import tpu as pltpu
from jax.experimental.pallas import tpu_sc as plsc

N_NODES = 10000
D = 128
E = 320000
N_PAD = 10112
ROWS_PER_TILE = N_PAD // 16
N_PAD2 = 10240
TCB = 1280
TCG = N_PAD2 // TCB
QB = TCB // D
NW = 32
EPW = E // NW
CHUNK = 64
NCHUNK = 160
EPW_P = NCHUNK * CHUNK
NBUF = 4
G = 40
NG = NCHUNK // G

_MESH = plsc.VectorSubcoreMesh(core_axis_name="c", subcore_axis_name="s")
_SC_PARAMS = pltpu.CompilerParams(needs_layout_passes=False)


@functools.partial(
    pl.kernel,
    out_type=jax.ShapeDtypeStruct((NW, N_PAD2), jnp.float32),
    mesh=_MESH,
    scratch_types=[
        pltpu.VMEM((EPW,), jnp.int32),
        pltpu.VMEM((N_PAD2,), jnp.float32),
    ],
    compiler_params=_SC_PARAMS,
)
def _deg_kernel(dst_hbm, hist_out, dst_v, hist):
    c = lax.axis_index("c")
    s = lax.axis_index("s")
    wid = c * 16 + s
    pltpu.sync_copy(dst_hbm.at[wid], dst_v)

    z = jnp.zeros((16,), jnp.float32)

    def zero_body(i, _):
        hist[pl.ds(i * 16, 16)] = z
        return ()

    lax.fori_loop(0, N_PAD2 // 16, zero_body, ())

    ones = jnp.ones((16,), jnp.float32)

    def scat_body(i, _):
        idx = dst_v[pl.ds(i * 16, 16)]
        plsc.addupdate_scatter(hist, [idx], ones)
        return ()

    lax.fori_loop(0, EPW // 16, scat_body, ())
    pltpu.sync_copy(hist, hist_out.at[wid])


def _xw_body(x_ref, w_ref, deg_ref, y_ref, dinv_ref):
    degs = jnp.sum(deg_ref[:, :, 0, :], axis=0) + 1.0
    dinv = lax.rsqrt(degs)
    dmat = jnp.transpose(
        jnp.broadcast_to(dinv[:, None, :], (QB, D, D)), (0, 2, 1)
    ).reshape(TCB, D)
    xw = jnp.dot(x_ref[...], w_ref[...], preferred_element_type=jnp.float32)
    y_ref[...] = xw * dmat
    dinv_ref[...] = dmat


def _xw_call(x, W, deg4):
    return pl.pallas_call(
        _xw_body,
        grid=(TCG,),
        in_specs=[
            pl.BlockSpec((TCB, D), lambda i: (i, 0)),
            pl.BlockSpec((D, D), lambda i: (0, 0)),
            pl.BlockSpec((NW, QB, 1, D), lambda i: (0, i, 0, 0)),
        ],
        out_specs=[
            pl.BlockSpec((TCB, D), lambda i: (i, 0)),
            pl.BlockSpec((TCB, D), lambda i: (i, 0)),
        ],
        out_shape=[
            jax.ShapeDtypeStruct((N_PAD2, D), jnp.float32),
            jax.ShapeDtypeStruct((N_PAD2, D), jnp.float32),
        ],
        compiler_params=pltpu.CompilerParams(
            dimension_semantics=("parallel",)
        ),
    )(x, W, deg4)


@functools.partial(
    pl.kernel,
    out_type=jax.ShapeDtypeStruct((2, N_PAD, D), jnp.float32),
    mesh=_MESH,
    scratch_types=[
        pltpu.VMEM((G, CHUNK), jnp.int32),
        pltpu.VMEM((G, CHUNK), jnp.int32),
        pltpu.VMEM((CHUNK, D), jnp.float32),
        pltpu.VMEM((CHUNK, D), jnp.float32),
        pltpu.VMEM((CHUNK, D), jnp.float32),
        pltpu.VMEM((CHUNK, D), jnp.float32),
        pltpu.VMEM_SHARED((N_PAD, D), jnp.float32),
        pltpu.SemaphoreType.DMA,
        pltpu.SemaphoreType.DMA,
        pltpu.SemaphoreType.DMA,
        pltpu.SemaphoreType.DMA,
        pltpu.SemaphoreType.DMA,
        pltpu.SemaphoreType.DMA,
        pltpu.SemaphoreType.DMA,
        pltpu.SemaphoreType.DMA,
    ],
    compiler_params=_SC_PARAMS,
)
def _agg_kernel(y_hbm, src_hbm, dst_hbm, acc_out,
                src_v, dst_v, r0, r1, r2, r3, accum,
                g0, g1, g2, g3, s0, s1, s2, s3):
    c = lax.axis_index("c")
    s = lax.axis_index("s")
    wid = c * 16 + s
    bufs = (r0, r1, r2, r3)
    gsems = (g0, g1, g2, g3)
    ssems = (s0, s1, s2, s3)

    rslice = pl.ds(s * ROWS_PER_TILE, ROWS_PER_TILE)
    base = s * ROWS_PER_TILE

    @pl.when(c == 0)
    def _():
        pltpu.sync_copy(y_hbm.at[rslice], accum.at[rslice])

    @pl.when(c == 1)
    def _():
        z = jnp.zeros((16,), jnp.float32)

        def zb(i, _):
            for q in range(8):
                r0[i, pl.ds(q * 16, 16)] = z
            return ()

        lax.fori_loop(0, CHUNK, zb, ())
        nfull = ROWS_PER_TILE // CHUNK
        for t in range(nfull):
            pltpu.sync_copy(r0, accum.at[pl.ds(base + t * CHUNK, CHUNK)])
        rem = ROWS_PER_TILE - nfull * CHUNK
        if rem:
            pltpu.sync_copy(
                r0.at[pl.ds(0, rem)],
                accum.at[pl.ds(base + ROWS_PER_TILE - rem, rem)],
            )

    plsc.subcore_barrier()

    def _wait_g(b):
        pltpu.make_async_copy(y_hbm.at[src_v.at[0]], bufs[b], gsems[b]).wait()

    def _wait_s(b):
        pltpu.make_async_copy(bufs[b], accum.at[dst_v.at[0]], ssems[b]).wait()

    for g in range(NG):
        pltpu.sync_copy(src_hbm.at[wid, pl.ds(g * G, G)], src_v)
        pltpu.sync_copy(dst_hbm.at[wid, pl.ds(g * G, G)], dst_v)
        pltpu.async_copy(y_hbm.at[src_v.at[0]], bufs[0], gsems[0])
        pltpu.async_copy(y_hbm.at[src_v.at[1]], bufs[1], gsems[1])

        def quad_body(q, _):
            for b in range(NBUF):
                j = NBUF * q + b
                _wait_g(b)
                pltpu.async_copy(
                    bufs[b], accum.at[dst_v.at[j]], ssems[b], add=True
                )
                bn = (b + 2) % NBUF

                @pl.when(j >= 2)
                def _():
                    _wait_s(bn)

                @pl.when(j + 2 < G)
                def _():
                    pltpu.async_copy(
                        y_hbm.at[src_v.at[j + 2]], bufs[bn], gsems[bn]
                    )

            return ()

        lax.fori_loop(0, G // NBUF, quad_body, ())
        _wait_s((G - 2) % NBUF)
        _wait_s((G - 1) % NBUF)

    plsc.subcore_barrier()
    pltpu.sync_copy(accum.at[rslice], acc_out.at[c, rslice])


def _out_body(acc_ref, dinv_ref, b_ref, o_ref):
    ssum = acc_ref[0] + acc_ref[1]
    o_ref[...] = jnp.maximum(ssum * dinv_ref[...] + b_ref[...], 0.0)


def _out_call(acc, dinvb, b2):
    return pl.pallas_call(
        _out_body,
        grid=(TCG,),
        in_specs=[
            pl.BlockSpec((2, TCB, D), lambda i: (0, i, 0)),
            pl.BlockSpec((TCB, D), lambda i: (i, 0)),
            pl.BlockSpec((1, D), lambda i: (0, 0)),
        ],
        out_specs=pl.BlockSpec((TCB, D), lambda i: (i, 0)),
        out_shape=jax.ShapeDtypeStruct((N_NODES, D), jnp.float32),
        compiler_params=pltpu.CompilerParams(
            dimension_semantics=("parallel",)
        ),
    )(acc, dinvb, b2)


def kernel(x, edge_index, W, b):
    src = edge_index[0].astype(jnp.int32)
    dst = edge_index[1].astype(jnp.int32)
    dst2 = dst.reshape(NW, EPW)
    npad = EPW_P - EPW
    pad_src = jnp.zeros((NW, npad), jnp.int32)
    pad_dst = jnp.broadcast_to(
        N_NODES + (jnp.arange(npad, dtype=jnp.int32) % (N_PAD - N_NODES)),
        (NW, npad),
    )
    src3 = jnp.concatenate([src.reshape(NW, EPW), pad_src], axis=1).reshape(
        NW, NCHUNK, CHUNK
    )
    dst3 = jnp.concatenate([dst2, pad_dst], axis=1).reshape(
        NW, NCHUNK, CHUNK
    )

    hist = _deg_kernel(dst2)
    deg4 = hist.reshape(NW, TCG * QB, 1, D)
    y, dinvb = _xw_call(x, W, deg4)
    acc = _agg_kernel(y, src3, dst3)
    return _out_call(acc, dinvb, b.reshape(1, D))


# speedup vs baseline: 2.4544x; 2.4544x over previous
import functools

import jax
import jax.numpy as jnp
from jax import lax
from jax.experimental import pallas as pl
from jax.experimental.pallas import tpu as pltpu
from jax.experimental.pallas import tpu_sc as plsc

N_NODES = 10000
D = 128
E = 320000
N_PAD = 10112
ROWS_PER_TILE = N_PAD // 16
N_PAD2 = 10240
TCB = 1280
TCG = N_PAD2 // TCB
QB = TCB // D
NW = 32
EPW = E // NW
CHUNK = 125
NCHUNK = EPW // CHUNK
G = 40
NG = NCHUNK // G

_MESH = plsc.VectorSubcoreMesh(core_axis_name="c", subcore_axis_name="s")
_SC_PARAMS = pltpu.CompilerParams(needs_layout_passes=False)


@functools.partial(
    pl.kernel,
    out_type=jax.ShapeDtypeStruct((NW, N_PAD2), jnp.float32),
    mesh=_MESH,
    scratch_types=[
        pltpu.VMEM((EPW,), jnp.int32),
        pltpu.VMEM((N_PAD2,), jnp.float32),
    ],
    compiler_params=_SC_PARAMS,
)
def _deg_kernel(dst_hbm, hist_out, dst_v, hist):
    c = lax.axis_index("c")
    s = lax.axis_index("s")
    wid = c * 16 + s
    pltpu.sync_copy(dst_hbm.at[wid], dst_v)

    z = jnp.zeros((16,), jnp.float32)

    def zero_body(i, _):
        hist[pl.ds(i * 16, 16)] = z
        return ()

    lax.fori_loop(0, N_PAD2 // 16, zero_body, ())

    ones = jnp.ones((16,), jnp.float32)

    def scat_body(i, _):
        idx = dst_v[pl.ds(i * 16, 16)]
        plsc.addupdate_scatter(hist, [idx], ones)
        return ()

    lax.fori_loop(0, EPW // 16, scat_body, ())
    pltpu.sync_copy(hist, hist_out.at[wid])


def _xw_body(x_ref, w_ref, deg_ref, y_ref, dinv_ref):
    degs = jnp.sum(deg_ref[:, :, 0, :], axis=0) + 1.0
    dinv = lax.rsqrt(degs)
    dmat = jnp.transpose(
        jnp.broadcast_to(dinv[:, None, :], (QB, D, D)), (0, 2, 1)
    ).reshape(TCB, D)
    xw = jnp.dot(x_ref[...], w_ref[...], preferred_element_type=jnp.float32)
    y_ref[...] = xw * dmat
    dinv_ref[...] = dmat


def _xw_call(x, W, deg4):
    return pl.pallas_call(
        _xw_body,
        grid=(TCG,),
        in_specs=[
            pl.BlockSpec((TCB, D), lambda i: (i, 0)),
            pl.BlockSpec((D, D), lambda i: (0, 0)),
            pl.BlockSpec((NW, QB, 1, D), lambda i: (0, i, 0, 0)),
        ],
        out_specs=[
            pl.BlockSpec((TCB, D), lambda i: (i, 0)),
            pl.BlockSpec((TCB, D), lambda i: (i, 0)),
        ],
        out_shape=[
            jax.ShapeDtypeStruct((N_PAD2, D), jnp.float32),
            jax.ShapeDtypeStruct((N_PAD2, D), jnp.float32),
        ],
        compiler_params=pltpu.CompilerParams(
            dimension_semantics=("parallel",)
        ),
    )(x, W, deg4)


@functools.partial(
    pl.kernel,
    out_type=jax.ShapeDtypeStruct((2, N_PAD, D), jnp.float32),
    mesh=_MESH,
    scratch_types=[
        pltpu.VMEM((G, CHUNK), jnp.int32),
        pltpu.VMEM((G, CHUNK), jnp.int32),
        pltpu.VMEM((CHUNK, D), jnp.float32),
        pltpu.VMEM((CHUNK, D), jnp.float32),
        pltpu.VMEM_SHARED((N_PAD, D), jnp.float32),
        pltpu.SemaphoreType.DMA,
        pltpu.SemaphoreType.DMA,
    ],
    compiler_params=_SC_PARAMS,
)
def _agg_kernel(y_hbm, src_hbm, dst_hbm, acc_out,
                src_v, dst_v, rows_a, rows_b, accum, sem_a, sem_b):
    c = lax.axis_index("c")
    s = lax.axis_index("s")
    wid = c * 16 + s

    rslice = pl.ds(s * ROWS_PER_TILE, ROWS_PER_TILE)
    base = s * ROWS_PER_TILE

    @pl.when(c == 0)
    def _():
        pltpu.sync_copy(y_hbm.at[rslice], accum.at[rslice])

    @pl.when(c == 1)
    def _():
        z = jnp.zeros((16,), jnp.float32)

        def zb(i, _):
            for q in range(8):
                rows_a[i, pl.ds(q * 16, 16)] = z
            return ()

        lax.fori_loop(0, CHUNK, zb, ())
        nfull = ROWS_PER_TILE // CHUNK
        for t in range(nfull):
            pltpu.sync_copy(rows_a, accum.at[pl.ds(base + t * CHUNK, CHUNK)])
        rem = ROWS_PER_TILE - nfull * CHUNK
        if rem:
            pltpu.sync_copy(
                rows_a.at[pl.ds(0, rem)],
                accum.at[pl.ds(base + ROWS_PER_TILE - rem, rem)],
            )

    plsc.subcore_barrier()

    for g in range(NG):
        pltpu.sync_copy(src_hbm.at[wid, pl.ds(g * G, G)], src_v)
        pltpu.sync_copy(dst_hbm.at[wid, pl.ds(g * G, G)], dst_v)
        pltpu.async_copy(y_hbm.at[src_v.at[0]], rows_a, sem_a)

        def pair_body(p, _):
            ja = 2 * p
            jb = 2 * p + 1
            pltpu.make_async_copy(y_hbm.at[src_v.at[0]], rows_a, sem_a).wait()
            pltpu.async_copy(y_hbm.at[src_v.at[jb]], rows_b, sem_b)
            pltpu.sync_copy(rows_a, accum.at[dst_v.at[ja]], add=True)
            pltpu.make_async_copy(y_hbm.at[src_v.at[0]], rows_b, sem_b).wait()

            @pl.when(p < G // 2 - 1)
            def _():
                pltpu.async_copy(y_hbm.at[src_v.at[ja + 2]], rows_a, sem_a)

            pltpu.sync_copy(rows_b, accum.at[dst_v.at[jb]], add=True)
            return ()

        lax.fori_loop(0, G // 2, pair_body, ())

    plsc.subcore_barrier()
    pltpu.sync_copy(accum.at[rslice], acc_out.at[c, rslice])


def _out_body(acc_ref, dinv_ref, b_ref, o_ref):
    ssum = acc_ref[0] + acc_ref[1]
    o_ref[...] = jnp.maximum(ssum * dinv_ref[...] + b_ref[...], 0.0)


def _out_call(acc, dinvb, b2):
    return pl.pallas_call(
        _out_body,
        grid=(TCG,),
        in_specs=[
            pl.BlockSpec((2, TCB, D), lambda i: (0, i, 0)),
            pl.BlockSpec((TCB, D), lambda i: (i, 0)),
            pl.BlockSpec((1, D), lambda i: (0, 0)),
        ],
        out_specs=pl.BlockSpec((TCB, D), lambda i: (i, 0)),
        out_shape=jax.ShapeDtypeStruct((N_NODES, D), jnp.float32),
        compiler_params=pltpu.CompilerParams(
            dimension_semantics=("parallel",)
        ),
    )(acc, dinvb, b2)


def kernel(x, edge_index, W, b):
    src = edge_index[0].astype(jnp.int32)
    dst = edge_index[1].astype(jnp.int32)
    dst2 = dst.reshape(NW, EPW)
    src3 = src.reshape(NW, NCHUNK, CHUNK)
    dst3 = dst.reshape(NW, NCHUNK, CHUNK)

    hist = _deg_kernel(dst2)
    deg4 = hist.reshape(NW, TCG * QB, 1, D)
    y, dinvb = _xw_call(x, W, deg4)
    acc = _agg_kernel(y, src3, dst3)
    return _out_call(acc, dinvb, b.reshape(1, D))
